# baseline (speedup 1.0000x reference)
import jax
import jax.numpy as jnp
from jax.experimental import pallas as pl
from jax.experimental.pallas import tpu as pltpu

INPUT_DIM = 1024
OUTPUT_DIM = 1024
NUM_EXPERTS = 8
TOPK = 2
TOKENS = 8192

BT = 2048
BS = 512
BG = 2048


def _prep_body(x_ref, wg_ref, bg_ref, we_ref, w_ref, web_ref):
    t = pl.program_id(0)
    web_ref[...] = we_ref[...].astype(jnp.bfloat16)
    x = x_ref[...]
    logits = jnp.dot(x, wg_ref[...], preferred_element_type=jnp.float32)
    logits = logits + bg_ref[...]
    probs = jax.nn.softmax(logits, axis=-1)
    rank = jnp.zeros(probs.shape, dtype=jnp.int32)
    idx = jax.lax.broadcasted_iota(jnp.int32, probs.shape, 1)
    for j in range(NUM_EXPERTS):
        pj = probs[:, j:j + 1]
        beat = (pj > probs) | ((pj == probs) & (j < idx))
        rank = rank + beat.astype(jnp.int32)
    w_ref[...] = jnp.where(rank < TOPK, probs, 0.0)


def _moe_body(x_ref, w_ref, we_ref, be_ref, o_ref):
    xb = x_ref[...].astype(jnp.bfloat16)
    w = w_ref[...]
    for s in range(BT // BS):
        sl = slice(s * BS, (s + 1) * BS)
        o_ref[sl, :] = jnp.dot(w[sl, :], be_ref[...],
                               preferred_element_type=jnp.float32)
    for e in range(NUM_EXPERTS):
        we = we_ref[:, e * OUTPUT_DIM:(e + 1) * OUTPUT_DIM]
        for s in range(BT // BS):
            sl = slice(s * BS, (s + 1) * BS)
            y = jnp.dot(xb[sl, :], we, preferred_element_type=jnp.float32)
            o_ref[sl, :] += w[sl, e:e + 1] * y


@jax.jit
def kernel(x, W_experts, b_experts, W_gate, b_gate):
    bg = b_gate.reshape(1, NUM_EXPERTS)
    be = b_experts.reshape(NUM_EXPERTS, OUTPUT_DIM)
    nt = TOKENS // BG
    w_all, we_bf16 = pl.pallas_call(
        _prep_body,
        grid=(nt,),
        in_specs=[
            pl.BlockSpec((BG, INPUT_DIM), lambda t: (t, 0)),
            pl.BlockSpec((INPUT_DIM, NUM_EXPERTS), lambda t: (0, 0)),
            pl.BlockSpec((1, NUM_EXPERTS), lambda t: (0, 0)),
            pl.BlockSpec((INPUT_DIM, NUM_EXPERTS * OUTPUT_DIM // 4), lambda t: (0, t)),
        ],
        out_specs=[
            pl.BlockSpec((BG, NUM_EXPERTS), lambda t: (t, 0)),
            pl.BlockSpec((INPUT_DIM, NUM_EXPERTS * OUTPUT_DIM // 4), lambda t: (0, t)),
        ],
        out_shape=[
            jax.ShapeDtypeStruct((TOKENS, NUM_EXPERTS), jnp.float32),
            jax.ShapeDtypeStruct((INPUT_DIM, NUM_EXPERTS * OUTPUT_DIM), jnp.bfloat16),
        ],
    )(x, W_gate, bg, W_experts)

    return pl.pallas_call(
        _moe_body,
        grid=(TOKENS // BT,),
        in_specs=[
            pl.BlockSpec((BT, INPUT_DIM), lambda t: (t, 0)),
            pl.BlockSpec((BT, NUM_EXPERTS), lambda t: (t, 0)),
            pl.BlockSpec((INPUT_DIM, NUM_EXPERTS * OUTPUT_DIM), lambda t: (0, 0)),
            pl.BlockSpec((NUM_EXPERTS, OUTPUT_DIM), lambda t: (0, 0)),
        ],
        out_specs=pl.BlockSpec((BT, OUTPUT_DIM), lambda t: (t, 0)),
        out_shape=jax.ShapeDtypeStruct((TOKENS, OUTPUT_DIM), jnp.float32),
        compiler_params=pltpu.CompilerParams(vmem_limit_bytes=63 * 1024 * 1024),
    )(x, w_all, we_bf16, be)
